# baseline (device time: 8011 ns/iter reference)
import jax
import jax.numpy as jnp
from jax import lax
from jax.experimental import pallas as pl
from jax.experimental.pallas import tpu as pltpu

N_DEV = 4
K = 8


def kernel(x):
    m, n = x.shape
    chunk = m // K

    def body(x_ref, out_ref, acc_ref, comm_ref, send_sems, recv_sems):
        my = lax.axis_index("i")
        pi = pl.program_id(0)
        barrier_sem = pltpu.get_barrier_semaphore()

        @pl.when(pi == 0)
        def _():
            for d in range(1, N_DEV):
                peer = lax.rem(my + d, N_DEV)
                pl.semaphore_signal(
                    barrier_sem, inc=1,
                    device_id=(peer,), device_id_type=pl.DeviceIdType.MESH,
                )

        blk = x_ref[...]
        cmax = blk[0:8, :]
        for g in range(1, chunk // 8):
            cmax = jnp.maximum(cmax, blk[g * 8 : (g + 1) * 8, :])

        @pl.when(pi == 0)
        def _():
            acc_ref[...] = cmax

        @pl.when(pi > 0)
        def _():
            acc_ref[...] = jnp.maximum(acc_ref[...], cmax)

        @pl.when(pi == K - 1)
        def _():
            comm_ref[pl.ds(my, 1), :, :] = jnp.max(
                acc_ref[...], axis=0, keepdims=True
            )[None]
            pl.semaphore_wait(barrier_sem, N_DEV - 1)

            sends = []
            for d in range(1, N_DEV):
                peer = lax.rem(my + d, N_DEV)
                rdma = pltpu.make_async_remote_copy(
                    src_ref=comm_ref.at[my],
                    dst_ref=comm_ref.at[my],
                    send_sem=send_sems.at[d - 1],
                    recv_sem=recv_sems.at[my],
                    device_id=(peer,),
                    device_id_type=pl.DeviceIdType.MESH,
                )
                rdma.start()
                sends.append(rdma)

            for d in range(1, N_DEV):
                src = lax.rem(my + d, N_DEV)
                recv = pltpu.make_async_remote_copy(
                    src_ref=comm_ref.at[src],
                    dst_ref=comm_ref.at[src],
                    send_sem=send_sems.at[d - 1],
                    recv_sem=recv_sems.at[src],
                    device_id=(src,),
                    device_id_type=pl.DeviceIdType.MESH,
                )
                recv.wait_recv()

            for rdma in sends:
                rdma.wait_send()

            out_ref[...] = jnp.max(comm_ref[...], axis=0)

    return pl.pallas_call(
        body,
        grid=(K,),
        out_shape=jax.ShapeDtypeStruct((1, n), jnp.float32),
        in_specs=[pl.BlockSpec((chunk, n), lambda i: (i, 0))],
        out_specs=pl.BlockSpec((1, n), lambda i: (0, 0)),
        scratch_shapes=[
            pltpu.VMEM((8, n), jnp.float32),
            pltpu.VMEM((N_DEV, 1, n), jnp.float32),
            pltpu.SemaphoreType.DMA((N_DEV - 1,)),
            pltpu.SemaphoreType.DMA((N_DEV,)),
        ],
        compiler_params=pltpu.CompilerParams(collective_id=0),
    )(x)


# device time: 7955 ns/iter; 1.0070x vs baseline; 1.0070x over previous
import jax
import jax.numpy as jnp
from jax import lax
from jax.experimental import pallas as pl
from jax.experimental.pallas import tpu as pltpu

N_DEV = 4
K = 8


def kernel(x):
    m, n = x.shape
    chunk = m // K

    def body(x_ref, out_ref, acc_ref, comm_ref, row_ref, send_sems, recv_sems, out_sem):
        my = lax.axis_index("i")
        pi = pl.program_id(0)
        barrier_sem = pltpu.get_barrier_semaphore()

        @pl.when(pi == 0)
        def _():
            for d in range(1, N_DEV):
                peer = lax.rem(my + d, N_DEV)
                pl.semaphore_signal(
                    barrier_sem, inc=1,
                    device_id=(peer,), device_id_type=pl.DeviceIdType.MESH,
                )

        blk = x_ref[...]
        cmax = blk[0:8, :]
        for g in range(1, chunk // 8):
            cmax = jnp.maximum(cmax, blk[g * 8 : (g + 1) * 8, :])

        @pl.when(pi == 0)
        def _():
            acc_ref[...] = cmax

        @pl.when(pi > 0)
        def _():
            acc_ref[...] = jnp.maximum(acc_ref[...], cmax)

        @pl.when(pi == K - 1)
        def _():
            comm_ref[pl.ds(my, 1), :, :] = jnp.max(
                acc_ref[...], axis=0, keepdims=True
            )[None]
            pl.semaphore_wait(barrier_sem, N_DEV - 1)

            sends = []
            for d in (2, 1, 3):
                peer = lax.rem(my + d, N_DEV)
                rdma = pltpu.make_async_remote_copy(
                    src_ref=comm_ref.at[my],
                    dst_ref=comm_ref.at[my],
                    send_sem=send_sems.at[d - 1],
                    recv_sem=recv_sems.at[my],
                    device_id=(peer,),
                    device_id_type=pl.DeviceIdType.MESH,
                )
                rdma.start()
                sends.append(rdma)

            for d in range(1, N_DEV):
                src = lax.rem(my + d, N_DEV)
                recv = pltpu.make_async_remote_copy(
                    src_ref=comm_ref.at[src],
                    dst_ref=comm_ref.at[src],
                    send_sem=send_sems.at[d - 1],
                    recv_sem=recv_sems.at[src],
                    device_id=(src,),
                    device_id_type=pl.DeviceIdType.MESH,
                )
                recv.wait_recv()

            for rdma in sends:
                rdma.wait_send()

            row_ref[...] = jnp.max(comm_ref[...], axis=0)
            cp = pltpu.make_async_copy(row_ref, out_ref, out_sem)
            cp.start()
            cp.wait()

    return pl.pallas_call(
        body,
        grid=(K,),
        out_shape=jax.ShapeDtypeStruct((1, n), jnp.float32),
        in_specs=[pl.BlockSpec((chunk, n), lambda i: (i, 0))],
        out_specs=pl.BlockSpec(memory_space=pl.MemorySpace.ANY),
        scratch_shapes=[
            pltpu.VMEM((8, n), jnp.float32),
            pltpu.VMEM((N_DEV, 1, n), jnp.float32),
            pltpu.VMEM((1, n), jnp.float32),
            pltpu.SemaphoreType.DMA((N_DEV - 1,)),
            pltpu.SemaphoreType.DMA((N_DEV,)),
            pltpu.SemaphoreType.DMA,
        ],
        compiler_params=pltpu.CompilerParams(collective_id=0),
    )(x)
